# Initial kernel scaffold; baseline (speedup 1.0000x reference)
import functools

import jax
import jax.numpy as jnp
from jax import lax
from jax.experimental import pallas as pl
from jax.experimental.pallas import tpu as pltpu
from jax.experimental.pallas import tpu_sc as plsc

B = 16384
L = 50
D = 16
OUT = 8

NW = 32
BAGS_W = B // NW
CHUNK_BAGS = 64
ROWS_CHUNK = CHUNK_BAGS * L
G = 128
NG = ROWS_CHUNK // G
NCHUNK = BAGS_W // CHUNK_BAGS


@functools.partial(
    pl.kernel,
    out_type=jax.ShapeDtypeStruct((B, D), jnp.float32),
    scratch_types=[
        pltpu.VMEM((NG, G), jnp.int32),
        pltpu.VMEM((NG, G), jnp.int32),
        pltpu.VMEM((ROWS_CHUNK, D), jnp.float32),
        pltpu.VMEM((ROWS_CHUNK, D), jnp.float32),
        pltpu.VMEM((CHUNK_BAGS, D), jnp.float32),
        pltpu.SemaphoreType.DMA,
        pltpu.SemaphoreType.DMA,
    ],
    mesh=plsc.VectorSubcoreMesh(core_axis_name="c", subcore_axis_name="s"),
)
def _sc_bag_sums(idx_hbm, table_hbm, out_hbm, ib0, ib1, rb0, rb1, sv,
                 sem0, sem1):
    wid = lax.axis_index("c") * 16 + lax.axis_index("s")
    idxrow0 = wid * (NCHUNK * NG)
    bag0 = wid * BAGS_W

    def load_idx(c, ib):
        pltpu.sync_copy(idx_hbm.at[pl.ds(idxrow0 + c * NG, NG)], ib)

    def fire(ib, rb, sem):
        def fb(j, carry):
            pltpu.make_async_copy(
                table_hbm.at[ib.at[j]], rb.at[pl.ds(j * G, G)], sem
            ).start()
            return carry
        lax.fori_loop(0, NG, fb, 0)

    def drain(rb, sem):
        pltpu.make_async_copy(
            table_hbm.at[pl.ds(0, ROWS_CHUNK)], rb, sem
        ).wait()

    def reduce_chunk(rb):
        def bag(i, carry):
            base = i * L
            acc = rb[base]
            for j in range(1, L):
                acc = acc + rb[base + j]
            sv[i] = acc
            return carry
        lax.fori_loop(0, CHUNK_BAGS, bag, 0)

    def store(c):
        pltpu.sync_copy(
            sv, out_hbm.at[pl.ds(bag0 + c * CHUNK_BAGS, CHUNK_BAGS)]
        )

    load_idx(0, ib0)
    fire(ib0, rb0, sem0)
    for c in range(NCHUNK):
        cur_ib, cur_rb, cur_sem = (ib0, rb0, sem0) if c % 2 == 0 else (
            ib1, rb1, sem1)
        nxt_ib, nxt_rb, nxt_sem = (ib1, rb1, sem1) if c % 2 == 0 else (
            ib0, rb0, sem0)
        if c + 1 < NCHUNK:
            load_idx(c + 1, nxt_ib)
            fire(nxt_ib, nxt_rb, nxt_sem)
        drain(cur_rb, cur_sem)
        reduce_chunk(cur_rb)
        store(c)


def _tc_linear_body(x_ref, wt_ref, b_ref, o_ref):
    o_ref[...] = (
        jnp.dot(x_ref[...], wt_ref[...], preferred_element_type=jnp.float32)
        + b_ref[...]
    )


def _tc_linear(x, wt, b2):
    m = x.shape[0]
    bm = 2048
    return pl.pallas_call(
        _tc_linear_body,
        grid=(m // bm,),
        in_specs=[
            pl.BlockSpec((bm, D), lambda i: (i, 0)),
            pl.BlockSpec((D, OUT), lambda i: (0, 0)),
            pl.BlockSpec((1, OUT), lambda i: (0, 0)),
        ],
        out_specs=pl.BlockSpec((bm, OUT), lambda i: (i, 0)),
        out_shape=jax.ShapeDtypeStruct((m, OUT), jnp.float32),
    )(x, wt, b2)


def kernel(indices, offsets, table, W, b):
    idx2d = indices.reshape(B * L // G, G)
    sums = _sc_bag_sums(idx2d, table)
    wt = W.T.astype(jnp.float32) * (1.0 / L)
    return _tc_linear(sums, wt, b.reshape(1, OUT))


# speedup vs baseline: 230.0500x; 230.0500x over previous
import functools

import jax
import jax.numpy as jnp
from jax import lax
from jax.experimental import pallas as pl
from jax.experimental.pallas import tpu as pltpu
from jax.experimental.pallas import tpu_sc as plsc

B = 16384
L = 50
D = 16
OUT = 8

NW = 32
BAGS_W = B // NW
CHUNK_BAGS = 64
ROWS_CHUNK = CHUNK_BAGS * L
G = 128
NG = ROWS_CHUNK // G
NCHUNK = BAGS_W // CHUNK_BAGS


@functools.partial(
    pl.kernel,
    out_type=jax.ShapeDtypeStruct((B, D), jnp.float32),
    scratch_types=[
        pltpu.VMEM((NCHUNK * NG, G), jnp.int32),
        pltpu.VMEM((ROWS_CHUNK, D), jnp.float32),
        pltpu.VMEM((ROWS_CHUNK, D), jnp.float32),
        pltpu.VMEM((CHUNK_BAGS, D), jnp.float32),
        pltpu.SemaphoreType.DMA,
        pltpu.SemaphoreType.DMA,
    ],
    mesh=plsc.VectorSubcoreMesh(core_axis_name="c", subcore_axis_name="s"),
    compiler_params=pltpu.CompilerParams(use_tc_tiling_on_sc=False),
)
def _sc_bag_sums(idx_hbm, table_hbm, out_hbm, ib, rb0, rb1, sv,
                 sem0, sem1):
    wid = lax.axis_index("c") * 16 + lax.axis_index("s")
    idxrow0 = wid * (NCHUNK * NG)
    bag0 = wid * BAGS_W

    def fire(c, rb, sem):
        def fb(j, carry):
            pltpu.make_async_copy(
                table_hbm.at[ib.at[c * NG + j]], rb.at[pl.ds(j * G, G)], sem
            ).start()
            return carry
        lax.fori_loop(0, NG, fb, 0)

    def drain(rb, sem):
        pltpu.make_async_copy(
            table_hbm.at[pl.ds(0, ROWS_CHUNK)], rb, sem
        ).wait()

    def reduce_chunk(rb):
        def bag(i, carry):
            base = i * L
            acc = rb[base]
            for j in range(1, L):
                acc = acc + rb[base + j]
            sv[i] = acc
            return carry
        lax.fori_loop(0, CHUNK_BAGS, bag, 0)

    def store(c):
        pltpu.sync_copy(
            sv, out_hbm.at[pl.ds(bag0 + c * CHUNK_BAGS, CHUNK_BAGS)]
        )

    pltpu.sync_copy(idx_hbm.at[pl.ds(idxrow0, NCHUNK * NG)], ib)
    fire(0, rb0, sem0)
    for c in range(NCHUNK):
        cur_rb, cur_sem = (rb0, sem0) if c % 2 == 0 else (rb1, sem1)
        nxt_rb, nxt_sem = (rb1, sem1) if c % 2 == 0 else (rb0, sem0)
        if c + 1 < NCHUNK:
            fire(c + 1, nxt_rb, nxt_sem)
        drain(cur_rb, cur_sem)
        reduce_chunk(cur_rb)
        store(c)


def _tc_linear_body(x_ref, wt_ref, b_ref, o_ref):
    o_ref[...] = (
        jnp.dot(x_ref[...], wt_ref[...], preferred_element_type=jnp.float32)
        + b_ref[...]
    )


def _tc_linear(x, wt, b2):
    m = x.shape[0]
    bm = 2048
    return pl.pallas_call(
        _tc_linear_body,
        grid=(m // bm,),
        in_specs=[
            pl.BlockSpec((bm, D), lambda i: (i, 0)),
            pl.BlockSpec((D, OUT), lambda i: (0, 0)),
            pl.BlockSpec((1, OUT), lambda i: (0, 0)),
        ],
        out_specs=pl.BlockSpec((bm, OUT), lambda i: (i, 0)),
        out_shape=jax.ShapeDtypeStruct((m, OUT), jnp.float32),
    )(x, wt, b2)


def kernel(indices, offsets, table, W, b):
    idx2d = indices.reshape(B * L // G, G)
    sums = _sc_bag_sums(idx2d, table)
    wt = W.T.astype(jnp.float32) * (1.0 / L)
    return _tc_linear(sums, wt, b.reshape(1, OUT))


# speedup vs baseline: 235.0885x; 1.0219x over previous
import functools

import jax
import jax.numpy as jnp
from jax import lax
from jax.experimental import pallas as pl
from jax.experimental.pallas import tpu as pltpu
from jax.experimental.pallas import tpu_sc as plsc

B = 16384
L = 50
D = 16
OUT = 8

NW = 32
BAGS_W = B // NW
CHUNK_BAGS = 64
ROWS_CHUNK = CHUNK_BAGS * L
G = 128
NG = ROWS_CHUNK // G
NCHUNK = BAGS_W // CHUNK_BAGS


@functools.partial(
    pl.kernel,
    out_type=jax.ShapeDtypeStruct((B, D), jnp.float32),
    scratch_types=[
        pltpu.VMEM((NCHUNK * NG * G,), jnp.int32),
        pltpu.VMEM((ROWS_CHUNK, D), jnp.float32),
        pltpu.VMEM((ROWS_CHUNK, D), jnp.float32),
        pltpu.VMEM((CHUNK_BAGS, D), jnp.float32),
        pltpu.SemaphoreType.DMA,
        pltpu.SemaphoreType.DMA,
    ],
    mesh=plsc.VectorSubcoreMesh(core_axis_name="c", subcore_axis_name="s"),
    compiler_params=pltpu.CompilerParams(use_tc_tiling_on_sc=False),
)
def _sc_bag_sums(idx_hbm, table_hbm, out_hbm, ib, rb0, rb1, sv,
                 sem0, sem1):
    wid = lax.axis_index("c") * 16 + lax.axis_index("s")
    idx0 = wid * (NCHUNK * NG * G)
    bag0 = wid * BAGS_W

    def fire(c, rb, sem):
        def fb(j, carry):
            pltpu.make_async_copy(
                table_hbm.at[ib.at[pl.ds((c * NG + j) * G, G)]],
                rb.at[pl.ds(j * G, G)], sem
            ).start()
            return carry
        lax.fori_loop(0, NG, fb, 0)

    def drain(rb, sem):
        pltpu.make_async_copy(
            table_hbm.at[pl.ds(0, ROWS_CHUNK)], rb, sem
        ).wait()

    def reduce_chunk(rb):
        def bag(i, carry):
            base = i * L
            acc = rb[base]
            for j in range(1, L):
                acc = acc + rb[base + j]
            sv[i] = acc
            return carry
        lax.fori_loop(0, CHUNK_BAGS, bag, 0)

    def store(c):
        pltpu.sync_copy(
            sv, out_hbm.at[pl.ds(bag0 + c * CHUNK_BAGS, CHUNK_BAGS)]
        )

    pltpu.sync_copy(idx_hbm.at[pl.ds(idx0, NCHUNK * NG * G)], ib)
    fire(0, rb0, sem0)
    for c in range(NCHUNK):
        cur_rb, cur_sem = (rb0, sem0) if c % 2 == 0 else (rb1, sem1)
        nxt_rb, nxt_sem = (rb1, sem1) if c % 2 == 0 else (rb0, sem0)
        if c + 1 < NCHUNK:
            fire(c + 1, nxt_rb, nxt_sem)
        drain(cur_rb, cur_sem)
        reduce_chunk(cur_rb)
        store(c)


def _tc_linear_body(x_ref, wt_ref, b_ref, o_ref):
    o_ref[...] = (
        jnp.dot(x_ref[...], wt_ref[...], preferred_element_type=jnp.float32)
        + b_ref[...]
    )


def _tc_linear(x, wt, b2):
    m = x.shape[0]
    bm = 2048
    return pl.pallas_call(
        _tc_linear_body,
        grid=(m // bm,),
        in_specs=[
            pl.BlockSpec((bm, D), lambda i: (i, 0)),
            pl.BlockSpec((D, OUT), lambda i: (0, 0)),
            pl.BlockSpec((1, OUT), lambda i: (0, 0)),
        ],
        out_specs=pl.BlockSpec((bm, OUT), lambda i: (i, 0)),
        out_shape=jax.ShapeDtypeStruct((m, OUT), jnp.float32),
    )(x, wt, b2)


def kernel(indices, offsets, table, W, b):
    sums = _sc_bag_sums(indices, table)
    wt = W.T.astype(jnp.float32) * (1.0 / L)
    return _tc_linear(sums, wt, b.reshape(1, OUT))


# speedup vs baseline: 316.8573x; 1.3478x over previous
import functools

import jax
import jax.numpy as jnp
from jax import lax
from jax.experimental import pallas as pl
from jax.experimental.pallas import tpu as pltpu
from jax.experimental.pallas import tpu_sc as plsc

B = 16384
L = 50
V = 1000000
D = 16
OUT = 8

NW = 32
BAGS_W = B // NW
CHUNK_BAGS = 64
ROWS_CHUNK = CHUNK_BAGS * L
G = 128
NG = ROWS_CHUNK // G
NCHUNK = BAGS_W // CHUNK_BAGS


@functools.partial(
    pl.kernel,
    out_type=jax.ShapeDtypeStruct((B, D), jnp.float32),
    scratch_types=[
        pltpu.VMEM((NCHUNK * NG * G,), jnp.int32),
        pltpu.VMEM((ROWS_CHUNK, D), jnp.float32),
        pltpu.VMEM((ROWS_CHUNK, D), jnp.float32),
        pltpu.VMEM((CHUNK_BAGS, D), jnp.float32),
        pltpu.SemaphoreType.DMA,
        pltpu.SemaphoreType.DMA,
    ],
    mesh=plsc.VectorSubcoreMesh(core_axis_name="c", subcore_axis_name="s"),
    compiler_params=pltpu.CompilerParams(use_tc_tiling_on_sc=False),
)
def _sc_bag_sums(idx_hbm, table_hbm, out_hbm, ib, rb0, rb1, sv,
                 sem0, sem1):
    wid = lax.axis_index("c") * 16 + lax.axis_index("s")
    idx0 = wid * (NCHUNK * NG * G)
    bag0 = wid * BAGS_W

    def fire(c, rb, sem):
        def fb(j, carry):
            pltpu.make_async_copy(
                table_hbm.at[ib.at[pl.ds((c * NG + j) * G, G)]],
                rb.at[pl.ds(j * G, G)], sem
            ).start()
            return carry
        lax.fori_loop(0, NG, fb, 0)

    def drain(rb, sem):
        pltpu.make_async_copy(
            table_hbm.at[pl.ds(0, ROWS_CHUNK)], rb, sem
        ).wait()

    def reduce_chunk(rb):
        def bag(i, carry):
            base = i * L
            acc = rb[base]
            for j in range(1, L):
                acc = acc + rb[base + j]
            sv[i] = acc
            return carry
        lax.fori_loop(0, CHUNK_BAGS, bag, 0)

    def store(c):
        pltpu.sync_copy(
            sv, out_hbm.at[pl.ds(bag0 + c * CHUNK_BAGS, CHUNK_BAGS)]
        )

    pltpu.sync_copy(idx_hbm.at[pl.ds(idx0, NCHUNK * NG * G)], ib)
    fire(0, rb0, sem0)
    for c in range(NCHUNK):
        cur_rb, cur_sem = (rb0, sem0) if c % 2 == 0 else (rb1, sem1)
        nxt_rb, nxt_sem = (rb1, sem1) if c % 2 == 0 else (rb0, sem0)
        if c + 1 < NCHUNK:
            fire(c + 1, nxt_rb, nxt_sem)
        drain(cur_rb, cur_sem)
        reduce_chunk(cur_rb)
        store(c)


TBN = 8192
TGRID = (V + TBN - 1) // TBN


def _tc_untile_body(t_ref, o_ref):
    x = t_ref[...]
    c = x.T
    c3 = c.reshape(TBN // 8, 8, D)
    o128 = jnp.concatenate([c3[:, j, :] for j in range(8)], axis=1)
    o_ref[...] = o128.reshape(-1)


def _tc_untile(table_t):
    return pl.pallas_call(
        _tc_untile_body,
        grid=(TGRID,),
        in_specs=[pl.BlockSpec((D, TBN), lambda i: (0, i))],
        out_specs=pl.BlockSpec((TBN * D,), lambda i: (i,)),
        out_shape=jax.ShapeDtypeStruct((V * D,), jnp.float32),
    )(table_t)


def _tc_linear_body(x_ref, wt_ref, b_ref, o_ref):
    o_ref[...] = (
        jnp.dot(x_ref[...], wt_ref[...], preferred_element_type=jnp.float32)
        + b_ref[...]
    )


def _tc_linear(x, wt, b2):
    m = x.shape[0]
    bm = 2048
    return pl.pallas_call(
        _tc_linear_body,
        grid=(m // bm,),
        in_specs=[
            pl.BlockSpec((bm, D), lambda i: (i, 0)),
            pl.BlockSpec((D, OUT), lambda i: (0, 0)),
            pl.BlockSpec((1, OUT), lambda i: (0, 0)),
        ],
        out_specs=pl.BlockSpec((bm, OUT), lambda i: (i, 0)),
        out_shape=jax.ShapeDtypeStruct((m, OUT), jnp.float32),
    )(x, wt, b2)


def kernel(indices, offsets, table, W, b):
    table_lin = _tc_untile(table.T).reshape(V, D)
    sums = _sc_bag_sums(indices, table_lin)
    wt = W.T.astype(jnp.float32) * (1.0 / L)
    return _tc_linear(sums, wt, b.reshape(1, OUT))
